# baseline (speedup 1.0000x reference)
import functools

import jax
import jax.numpy as jnp
from jax import lax
from jax.experimental import pallas as pl
from jax.experimental.pallas import tpu as pltpu
from jax.experimental.pallas import tpu_sc as plsc

NUM_ACTIONS = 100000
D_MODEL = 128
HALF = D_MODEL // 2
B = 4096
L = 200
TOKENS = B * L

CHUNK_TOKENS = [102400, 102400, 204800, 204800, 204800]
assert sum(CHUNK_TOKENS) == TOKENS

NW = 32
CHUNK = 128

VPAD = NUM_ACTIONS + 1


def _fuse_table(tableT, Wo_top):
    RBLK = 2048
    grid = (pl.cdiv(VPAD, RBLK),)

    def body(t_ref, w_ref, o_ref):
        o_ref[...] = lax.dot_general(
            t_ref[...],
            w_ref[...],
            (((0,), (0,)), ((), ())),
            preferred_element_type=jnp.float32,
        )

    return pl.pallas_call(
        body,
        grid=grid,
        in_specs=[
            pl.BlockSpec((HALF, RBLK), lambda i: (0, i)),
            pl.BlockSpec((HALF, D_MODEL), lambda i: (0, 0)),
        ],
        out_specs=pl.BlockSpec((RBLK, D_MODEL), lambda i: (i, 0)),
        out_shape=jax.ShapeDtypeStruct((VPAD, D_MODEL), jnp.float32),
    )(tableT, Wo_top)


NBUF = 4
LOOK = 2


def _sc_gather(types32, table2, chtok):
    rows_per_w = chtok // NW
    CHUNKS = rows_per_w // CHUNK
    mesh = plsc.VectorSubcoreMesh(core_axis_name="c", subcore_axis_name="s")

    @functools.partial(
        pl.kernel,
        out_type=jax.ShapeDtypeStruct((chtok, D_MODEL), jnp.float32),
        mesh=mesh,
        scratch_types=[
            pltpu.VMEM((CHUNKS, CHUNK), jnp.int32),
            [pltpu.VMEM((CHUNK, D_MODEL), jnp.float32) for _ in range(NBUF)],
            [pltpu.SemaphoreType.DMA for _ in range(NBUF)],
            [pltpu.SemaphoreType.DMA for _ in range(NBUF)],
        ],
    )
    def gather_kernel(idx_hbm, table_hbm, out_hbm, idx_v, bufs, gsem, osem):
        wid = lax.axis_index("s") * 2 + lax.axis_index("c")
        base = wid * rows_per_w
        pltpu.sync_copy(idx_hbm.at[wid], idx_v)

        def gath(j, s):
            pltpu.async_copy(table_hbm.at[idx_v.at[j]], bufs[s], gsem[s])

        def gath_wait(j, s):
            pltpu.make_async_copy(
                table_hbm.at[idx_v.at[j]], bufs[s], gsem[s]
            ).wait()

        def outc(j, s):
            pltpu.async_copy(
                bufs[s], out_hbm.at[pl.ds(base + j * CHUNK, CHUNK)], osem[s]
            )

        def outc_wait(j, s):
            pltpu.make_async_copy(
                bufs[s], out_hbm.at[pl.ds(base + j * CHUNK, CHUNK)], osem[s]
            ).wait()

        def prefetch(x, s):
            if isinstance(x, int) and x < NBUF:
                pass
            else:
                outc_wait(x - NBUF, s)
            gath(x, s)

        def process(j, s):
            gath_wait(j, s)
            outc(j, s)

        for j in range(LOOK):
            prefetch(j, j % NBUF)
        for j in range(LOOK):
            prefetch(j + LOOK, (j + LOOK) % NBUF)
            process(j, j % NBUF)

        G = (CHUNKS - 2 * LOOK) // NBUF
        def body(o, _):
            for k in range(NBUF):
                j = LOOK + o * NBUF + k
                prefetch(j + LOOK, k)
                process(j, (LOOK + k) % NBUF)
            return 0

        lax.fori_loop(0, G, body, 0)

        for j in range(LOOK + G * NBUF, CHUNKS):
            if j + LOOK < CHUNKS:
                prefetch(j + LOOK, (j + LOOK) % NBUF)
            process(j, j % NBUF)
        for j in range(CHUNKS - NBUF, CHUNKS):
            outc_wait(j, j % NBUF)

    return gather_kernel(types32, table2)


BB = 16
TBLK = BB * L


def _tc_tail(e2, cont_t, Wc, bc2, Wo, bo2, gamma2, beta2, start, prev):
    nblk = e2.shape[0] // TBLK
    off = start // TBLK
    grid = (nblk,)

    def body(e_ref, c_ref, wc_ref, bc_ref, wo_ref, bo_ref, g_ref, b_ref,
             *rest):
        o_ref = rest[-1]
        wo_bot = wo_ref[...][HALF:, :]
        w2 = jnp.dot(wc_ref[...], wo_bot, preferred_element_type=jnp.float32)
        b2 = (
            jnp.dot(bc_ref[...], wo_bot, preferred_element_type=jnp.float32)
            + bo_ref[...]
        )
        gam = g_ref[...]
        bet = b_ref[...]
        for bb in range(BB):
            o = (
                e_ref[pl.ds(bb * L, L), :]
                + lax.dot_general(
                    c_ref[:, bb, :],
                    w2,
                    (((0,), (0,)), ((), ())),
                    preferred_element_type=jnp.float32,
                )
                + b2
            )
            mu = jnp.mean(o, axis=-1, keepdims=True)
            d = o - mu
            var = jnp.mean(d * d, axis=-1, keepdims=True)
            y = d * lax.rsqrt(var + 1e-5)
            o_ref[pl.ds(bb * L, L), :] = y * gam + bet

    def wspec(shape):
        return pl.BlockSpec(shape, lambda i: (0, 0))

    in_specs = [
        pl.BlockSpec((TBLK, D_MODEL), lambda i: (i, 0)),
        pl.BlockSpec((3, BB, L), lambda i, c=off: (0, i + c, 0)),
        wspec((3, HALF)),
        wspec((1, HALF)),
        wspec((D_MODEL, D_MODEL)),
        wspec((1, D_MODEL)),
        wspec((1, D_MODEL)),
        wspec((1, D_MODEL)),
    ]
    args = [e2, cont_t, Wc, bc2, Wo, bo2, gamma2, beta2]
    io_aliases = {}
    if prev is not None:
        in_specs.append(pl.BlockSpec(memory_space=pl.ANY))
        args.append(prev)
        io_aliases = {8: 0}

    return pl.pallas_call(
        body,
        grid=grid,
        in_specs=in_specs,
        out_specs=pl.BlockSpec(
            (TBLK, D_MODEL), lambda i, c=off: (i + c, 0)
        ),
        out_shape=jax.ShapeDtypeStruct((TOKENS, D_MODEL), jnp.float32),
        input_output_aliases=io_aliases,
    )(*args)


def kernel(types, cont, table, Wc, bc, Wo, bo, gamma, beta):
    types_flat = types.astype(jnp.int32).reshape(TOKENS)
    table2 = _fuse_table(table.T, Wo[:HALF, :])
    cont_t = jnp.transpose(cont, (2, 0, 1))
    bc2 = bc.reshape(1, HALF)
    bo2 = bo.reshape(1, D_MODEL)
    gamma2 = gamma.reshape(1, D_MODEL)
    beta2 = beta.reshape(1, D_MODEL)

    starts = [sum(CHUNK_TOKENS[:c]) for c in range(len(CHUNK_TOKENS))]
    e2s = []
    for c, chtok in enumerate(CHUNK_TOKENS):
        idx = types_flat[starts[c] : starts[c] + chtok].reshape(
            NW, chtok // NW // CHUNK, CHUNK
        )
        e2s.append(_sc_gather(idx, table2, chtok))
    out = None
    for c, chtok in enumerate(CHUNK_TOKENS):
        out = _tc_tail(
            e2s[c], cont_t, Wc, bc2, Wo, bo2, gamma2, beta2, starts[c], out
        )
    return out.reshape(B, L, D_MODEL)


# speedup vs baseline: 1.3547x; 1.3547x over previous
import functools

import jax
import jax.numpy as jnp
from jax import lax
from jax.experimental import pallas as pl
from jax.experimental.pallas import tpu as pltpu
from jax.experimental.pallas import tpu_sc as plsc

NUM_ACTIONS = 100000
D_MODEL = 128
HALF = D_MODEL // 2
B = 4096
L = 200
TOKENS = B * L

CHUNK_TOKENS = [204800, 204800, 204800, 204800]
assert sum(CHUNK_TOKENS) == TOKENS

NW = 32
CHUNK = 128

VPAD = NUM_ACTIONS + 1


def _fuse_table(tableT, Wo_top):
    RBLK = 2048
    grid = (pl.cdiv(VPAD, RBLK),)

    def body(t_ref, w_ref, o_ref):
        o_ref[...] = lax.dot_general(
            t_ref[...],
            w_ref[...],
            (((0,), (0,)), ((), ())),
            preferred_element_type=jnp.float32,
        )

    return pl.pallas_call(
        body,
        grid=grid,
        in_specs=[
            pl.BlockSpec((HALF, RBLK), lambda i: (0, i)),
            pl.BlockSpec((HALF, D_MODEL), lambda i: (0, 0)),
        ],
        out_specs=pl.BlockSpec((RBLK, D_MODEL), lambda i: (i, 0)),
        out_shape=jax.ShapeDtypeStruct((VPAD, D_MODEL), jnp.float32),
    )(tableT, Wo_top)


NBUF = 4
LOOK = 2


def _sc_gather(types32, table2, chtok):
    rows_per_w = chtok // NW
    CHUNKS = rows_per_w // CHUNK
    mesh = plsc.VectorSubcoreMesh(core_axis_name="c", subcore_axis_name="s")

    @functools.partial(
        pl.kernel,
        out_type=jax.ShapeDtypeStruct((chtok, D_MODEL), jnp.float32),
        mesh=mesh,
        scratch_types=[
            pltpu.VMEM((CHUNKS, CHUNK), jnp.int32),
            [pltpu.VMEM((CHUNK, D_MODEL), jnp.float32) for _ in range(NBUF)],
            [pltpu.SemaphoreType.DMA for _ in range(NBUF)],
            [pltpu.SemaphoreType.DMA for _ in range(NBUF)],
        ],
    )
    def gather_kernel(idx_hbm, table_hbm, out_hbm, idx_v, bufs, gsem, osem):
        wid = lax.axis_index("s") * 2 + lax.axis_index("c")
        base = wid * rows_per_w
        pltpu.sync_copy(idx_hbm.at[wid], idx_v)

        def gath(j, s):
            pltpu.async_copy(table_hbm.at[idx_v.at[j]], bufs[s], gsem[s])

        def gath_wait(j, s):
            pltpu.make_async_copy(
                table_hbm.at[idx_v.at[j]], bufs[s], gsem[s]
            ).wait()

        def outc(j, s):
            pltpu.async_copy(
                bufs[s], out_hbm.at[pl.ds(base + j * CHUNK, CHUNK)], osem[s]
            )

        def outc_wait(j, s):
            pltpu.make_async_copy(
                bufs[s], out_hbm.at[pl.ds(base + j * CHUNK, CHUNK)], osem[s]
            ).wait()

        def prefetch(x, s):
            if isinstance(x, int) and x < NBUF:
                pass
            else:
                outc_wait(x - NBUF, s)
            gath(x, s)

        def process(j, s):
            gath_wait(j, s)
            outc(j, s)

        for j in range(LOOK):
            prefetch(j, j % NBUF)
        for j in range(LOOK):
            prefetch(j + LOOK, (j + LOOK) % NBUF)
            process(j, j % NBUF)

        G = (CHUNKS - 2 * LOOK) // NBUF
        def body(o, _):
            for k in range(NBUF):
                j = LOOK + o * NBUF + k
                prefetch(j + LOOK, k)
                process(j, (LOOK + k) % NBUF)
            return 0

        lax.fori_loop(0, G, body, 0)

        for j in range(LOOK + G * NBUF, CHUNKS):
            if j + LOOK < CHUNKS:
                prefetch(j + LOOK, (j + LOOK) % NBUF)
            process(j, j % NBUF)
        for j in range(CHUNKS - NBUF, CHUNKS):
            outc_wait(j, j % NBUF)

    return gather_kernel(types32, table2)


def _tc_tail(e2, cont3, Wc, bc2, Wo, bo2, gamma2, beta2, start, prev):
    TBLK = 4096
    nblk = e2.shape[0] // TBLK
    off = start // TBLK
    grid = (nblk,)

    def body(e_ref, c_ref, wc_ref, bc_ref, wo_ref, bo_ref, g_ref, b_ref,
             *rest):
        o_ref = rest[-1]
        wo_bot = wo_ref[...][HALF:, :]
        w2 = jnp.dot(wc_ref[...], wo_bot, preferred_element_type=jnp.float32)
        b2 = (
            jnp.dot(bc_ref[...], wo_bot, preferred_element_type=jnp.float32)
            + bo_ref[...]
        )
        o = (
            e_ref[...]
            + lax.dot_general(
                c_ref[...],
                w2,
                (((0,), (0,)), ((), ())),
                preferred_element_type=jnp.float32,
            )
            + b2
        )
        mu = jnp.mean(o, axis=-1, keepdims=True)
        d = o - mu
        var = jnp.mean(d * d, axis=-1, keepdims=True)
        y = d * lax.rsqrt(var + 1e-5)
        o_ref[...] = y * g_ref[...] + b_ref[...]

    def wspec(shape):
        return pl.BlockSpec(shape, lambda i: (0, 0))

    in_specs = [
        pl.BlockSpec((TBLK, D_MODEL), lambda i: (i, 0)),
        pl.BlockSpec((3, TBLK), lambda i, c=off: (0, i + c)),
        wspec((3, HALF)),
        wspec((1, HALF)),
        wspec((D_MODEL, D_MODEL)),
        wspec((1, D_MODEL)),
        wspec((1, D_MODEL)),
        wspec((1, D_MODEL)),
    ]
    args = [e2, cont3, Wc, bc2, Wo, bo2, gamma2, beta2]
    io_aliases = {}
    if prev is not None:
        in_specs.append(pl.BlockSpec(memory_space=pl.ANY))
        args.append(prev)
        io_aliases = {8: 0}

    return pl.pallas_call(
        body,
        grid=grid,
        in_specs=in_specs,
        out_specs=pl.BlockSpec(
            (TBLK, D_MODEL), lambda i, c=off: (i + c, 0)
        ),
        out_shape=jax.ShapeDtypeStruct((TOKENS, D_MODEL), jnp.float32),
        input_output_aliases=io_aliases,
    )(*args)


def kernel(types, cont, table, Wc, bc, Wo, bo, gamma, beta):
    types_flat = types.astype(jnp.int32).reshape(TOKENS)
    table2 = _fuse_table(table.T, Wo[:HALF, :])
    cont3 = jnp.transpose(cont, (2, 0, 1)).reshape(3, TOKENS)
    bc2 = bc.reshape(1, HALF)
    bo2 = bo.reshape(1, D_MODEL)
    gamma2 = gamma.reshape(1, D_MODEL)
    beta2 = beta.reshape(1, D_MODEL)

    starts = [sum(CHUNK_TOKENS[:c]) for c in range(len(CHUNK_TOKENS))]
    e2s = []
    for c, chtok in enumerate(CHUNK_TOKENS):
        idx = types_flat[starts[c] : starts[c] + chtok].reshape(
            NW, chtok // NW // CHUNK, CHUNK
        )
        e2s.append(_sc_gather(idx, table2, chtok))
    out = None
    for c, chtok in enumerate(CHUNK_TOKENS):
        out = _tc_tail(
            e2s[c], cont3, Wc, bc2, Wo, bo2, gamma2, beta2, starts[c], out
        )
    return out.reshape(B, L, D_MODEL)


# speedup vs baseline: 1.4599x; 1.0776x over previous
import functools

import jax
import jax.numpy as jnp
from jax import lax
from jax.experimental import pallas as pl
from jax.experimental.pallas import tpu as pltpu
from jax.experimental.pallas import tpu_sc as plsc

NUM_ACTIONS = 100000
D_MODEL = 128
HALF = D_MODEL // 2
B = 4096
L = 200
TOKENS = B * L

CHUNK_TOKENS = [204800, 204800, 204800, 204800]
assert sum(CHUNK_TOKENS) == TOKENS

NW = 32
CHUNK = 128

VPAD = NUM_ACTIONS + 1


def _fuse_table(tableT, Wo_top):
    RBLK = 4096
    grid = (pl.cdiv(VPAD, RBLK),)

    def body(t_ref, w_ref, o_ref):
        o_ref[...] = lax.dot_general(
            t_ref[...],
            w_ref[...],
            (((0,), (0,)), ((), ())),
            preferred_element_type=jnp.float32,
        )

    return pl.pallas_call(
        body,
        grid=grid,
        in_specs=[
            pl.BlockSpec((HALF, RBLK), lambda i: (0, i)),
            pl.BlockSpec((HALF, D_MODEL), lambda i: (0, 0)),
        ],
        out_specs=pl.BlockSpec((RBLK, D_MODEL), lambda i: (i, 0)),
        out_shape=jax.ShapeDtypeStruct((VPAD, D_MODEL), jnp.float32),
    )(tableT, Wo_top)


NBUF = 4
LOOK = 2


def _sc_gather(types32, table2, chtok):
    rows_per_w = chtok // NW
    CHUNKS = rows_per_w // CHUNK
    mesh = plsc.VectorSubcoreMesh(core_axis_name="c", subcore_axis_name="s")

    @functools.partial(
        pl.kernel,
        out_type=jax.ShapeDtypeStruct((chtok, D_MODEL), jnp.float32),
        mesh=mesh,
        scratch_types=[
            pltpu.VMEM((CHUNKS, CHUNK), jnp.int32),
            [pltpu.VMEM((CHUNK, D_MODEL), jnp.float32) for _ in range(NBUF)],
            [pltpu.SemaphoreType.DMA for _ in range(NBUF)],
            [pltpu.SemaphoreType.DMA for _ in range(NBUF)],
        ],
    )
    def gather_kernel(idx_hbm, table_hbm, out_hbm, idx_v, bufs, gsem, osem):
        wid = lax.axis_index("s") * 2 + lax.axis_index("c")
        base = wid * rows_per_w
        pltpu.sync_copy(idx_hbm.at[wid], idx_v)

        def gath(j, s):
            pltpu.async_copy(table_hbm.at[idx_v.at[j]], bufs[s], gsem[s])

        def gath_wait(j, s):
            pltpu.make_async_copy(
                table_hbm.at[idx_v.at[j]], bufs[s], gsem[s]
            ).wait()

        def outc(j, s):
            pltpu.async_copy(
                bufs[s], out_hbm.at[pl.ds(base + j * CHUNK, CHUNK)], osem[s]
            )

        def outc_wait(j, s):
            pltpu.make_async_copy(
                bufs[s], out_hbm.at[pl.ds(base + j * CHUNK, CHUNK)], osem[s]
            ).wait()

        def prefetch(x, s):
            if isinstance(x, int) and x < NBUF:
                pass
            else:
                outc_wait(x - NBUF, s)
            gath(x, s)

        def process(j, s):
            gath_wait(j, s)
            outc(j, s)

        for j in range(LOOK):
            prefetch(j, j % NBUF)
        for j in range(LOOK):
            prefetch(j + LOOK, (j + LOOK) % NBUF)
            process(j, j % NBUF)

        G = (CHUNKS - 2 * LOOK) // NBUF
        def body(o, _):
            for k in range(NBUF):
                j = LOOK + o * NBUF + k
                prefetch(j + LOOK, k)
                process(j, (LOOK + k) % NBUF)
            return 0

        lax.fori_loop(0, G, body, 0)

        for j in range(LOOK + G * NBUF, CHUNKS):
            if j + LOOK < CHUNKS:
                prefetch(j + LOOK, (j + LOOK) % NBUF)
            process(j, j % NBUF)
        for j in range(CHUNKS - NBUF, CHUNKS):
            outc_wait(j, j % NBUF)

    return gather_kernel(types32, table2)


def _tc_tail(e2, cont3, Wc, bc2, Wo, bo2, gamma2, beta2, start, prev):
    TBLK = 8192
    nblk = e2.shape[0] // TBLK
    off = start // TBLK
    grid = (nblk,)

    def body(e_ref, c_ref, wc_ref, bc_ref, wo_ref, bo_ref, g_ref, b_ref,
             *rest):
        o_ref = rest[-1]
        wo_bot = wo_ref[...][HALF:, :]
        w2 = jnp.dot(wc_ref[...], wo_bot, preferred_element_type=jnp.float32)
        b2 = (
            jnp.dot(bc_ref[...], wo_bot, preferred_element_type=jnp.float32)
            + bo_ref[...]
        )
        o = (
            e_ref[...]
            + lax.dot_general(
                c_ref[...],
                w2,
                (((0,), (0,)), ((), ())),
                preferred_element_type=jnp.float32,
            )
            + b2
        )
        mu = jnp.mean(o, axis=-1, keepdims=True)
        d = o - mu
        var = jnp.mean(d * d, axis=-1, keepdims=True)
        y = d * lax.rsqrt(var + 1e-5)
        o_ref[...] = y * g_ref[...] + b_ref[...]

    def wspec(shape):
        return pl.BlockSpec(shape, lambda i: (0, 0))

    in_specs = [
        pl.BlockSpec((TBLK, D_MODEL), lambda i: (i, 0)),
        pl.BlockSpec((3, TBLK), lambda i, c=off: (0, i + c)),
        wspec((3, HALF)),
        wspec((1, HALF)),
        wspec((D_MODEL, D_MODEL)),
        wspec((1, D_MODEL)),
        wspec((1, D_MODEL)),
        wspec((1, D_MODEL)),
    ]
    args = [e2, cont3, Wc, bc2, Wo, bo2, gamma2, beta2]
    io_aliases = {}
    if prev is not None:
        in_specs.append(pl.BlockSpec(memory_space=pl.ANY))
        args.append(prev)
        io_aliases = {8: 0}

    return pl.pallas_call(
        body,
        grid=grid,
        in_specs=in_specs,
        out_specs=pl.BlockSpec(
            (TBLK, D_MODEL), lambda i, c=off: (i + c, 0)
        ),
        out_shape=jax.ShapeDtypeStruct((TOKENS, D_MODEL), jnp.float32),
        input_output_aliases=io_aliases,
    )(*args)


def kernel(types, cont, table, Wc, bc, Wo, bo, gamma, beta):
    types_flat = types.astype(jnp.int32).reshape(TOKENS)
    table2 = _fuse_table(table.T, Wo[:HALF, :])
    cont3 = jnp.transpose(cont, (2, 0, 1)).reshape(3, TOKENS)
    bc2 = bc.reshape(1, HALF)
    bo2 = bo.reshape(1, D_MODEL)
    gamma2 = gamma.reshape(1, D_MODEL)
    beta2 = beta.reshape(1, D_MODEL)

    starts = [sum(CHUNK_TOKENS[:c]) for c in range(len(CHUNK_TOKENS))]
    e2s = []
    for c, chtok in enumerate(CHUNK_TOKENS):
        idx = types_flat[starts[c] : starts[c] + chtok].reshape(
            NW, chtok // NW // CHUNK, CHUNK
        )
        e2s.append(_sc_gather(idx, table2, chtok))
    out = None
    for c, chtok in enumerate(CHUNK_TOKENS):
        out = _tc_tail(
            e2s[c], cont3, Wc, bc2, Wo, bo2, gamma2, beta2, starts[c], out
        )
    return out.reshape(B, L, D_MODEL)
